# baseline (speedup 1.0000x reference)
import functools

import jax
import jax.numpy as jnp
from jax import lax
from jax.experimental import pallas as pl
from jax.experimental.pallas import tpu as pltpu
from jax.experimental.pallas import tpu_sc as plsc

B = 16384
D = 32
NC, NS, L = 2, 16, 16
NW = NC * NS
BPW = B // NW
NCHUNK = BPW // 128

_mesh = plsc.VectorSubcoreMesh(core_axis_name="c", subcore_axis_name="s")


@functools.partial(
    pl.kernel,
    out_type=[jax.ShapeDtypeStruct((B, D), jnp.float32)] * 4,
    mesh=_mesh,
    compiler_params=pltpu.CompilerParams(use_tc_tiling_on_sc=False),
    scratch_types=[
        pltpu.VMEM((NCHUNK, 128), jnp.int32),
        pltpu.VMEM((NCHUNK, 128), jnp.int32),
        pltpu.VMEM((BPW, D), jnp.float32),
        pltpu.VMEM((BPW, D), jnp.float32),
        pltpu.VMEM((BPW, D), jnp.float32),
        pltpu.VMEM((BPW, D), jnp.float32),
        pltpu.SemaphoreType.DMA,
    ],
)
def _sc_gather4(u_hbm, i_hbm, ug_hbm, ig_hbm, um_hbm, im_hbm,
                ug_out, ig_out, um_out, im_out,
                u_idx, i_idx, bug, big, bum, bim, sem):
    wid = lax.axis_index("s") * NC + lax.axis_index("c")
    base = wid * BPW
    pltpu.sync_copy(u_hbm.at[pl.ds(wid * NCHUNK, NCHUNK)], u_idx)
    pltpu.sync_copy(i_hbm.at[pl.ds(wid * NCHUNK, NCHUNK)], i_idx)
    copies = []
    for table, idx, buf in ((ug_hbm, u_idx, bug), (ig_hbm, i_idx, big),
                            (um_hbm, u_idx, bum), (im_hbm, i_idx, bim)):
        for j in range(NCHUNK):
            copies.append(pltpu.async_copy(
                table.at[idx.at[j]], buf.at[pl.ds(j * 128, 128)], sem))
    for c in copies:
        c.wait()
    pltpu.sync_copy(bug, ug_out.at[pl.ds(base, BPW)])
    pltpu.sync_copy(big, ig_out.at[pl.ds(base, BPW)])
    pltpu.sync_copy(bum, um_out.at[pl.ds(base, BPW)])
    pltpu.sync_copy(bim, im_out.at[pl.ds(base, BPW)])


BLK = 2048


def _tc_mlp(ug, ig, um, im, w1u, w1i, b1, w2, b2, w3, b3, whg, whx, bh, out):
    f32 = jnp.float32
    x1 = jnp.maximum(jnp.dot(um[...], w1u[...], preferred_element_type=f32)
                     + jnp.dot(im[...], w1i[...], preferred_element_type=f32)
                     + b1[...], 0.0)
    x2 = jnp.maximum(jnp.dot(x1, w2[...], preferred_element_type=f32) + b2[...], 0.0)
    x3 = jnp.maximum(jnp.dot(x2, w3[...], preferred_element_type=f32) + b3[...], 0.0)
    gmf = ug[...] * ig[...]
    out[...] = (jnp.dot(gmf, whg[...], preferred_element_type=f32)
                + jnp.dot(x3, whx[...], preferred_element_type=f32)
                + bh[...])


def kernel(u, i, user_gmf, item_gmf, user_mlp, item_mlp,
           W1, b1, W2, b2, W3, b3, Wh, bh):
    u2 = u.astype(jnp.int32).reshape(B // 128, 128)
    i2 = i.astype(jnp.int32).reshape(B // 128, 128)
    ug, ig, um, im = _sc_gather4(u2, i2, user_gmf, item_gmf, user_mlp, item_mlp)

    row = lambda bs: pl.BlockSpec(bs, lambda b: (b, 0))
    full = lambda bs: pl.BlockSpec(bs, lambda b: (0, 0))
    out2 = pl.pallas_call(
        _tc_mlp,
        grid=(B // BLK,),
        in_specs=[row((BLK, D))] * 4 + [
            full((D, 64)), full((D, 64)), full((1, 64)),
            full((64, 32)), full((1, 32)),
            full((32, 16)), full((1, 16)),
            full((D, 1)), full((16, 1)), full((1, 1)),
        ],
        out_specs=row((BLK, 1)),
        out_shape=jax.ShapeDtypeStruct((B, 1), jnp.float32),
    )(ug, ig, um, im,
      W1[:D], W1[D:], b1.reshape(1, 64),
      W2, b2.reshape(1, 32),
      W3, b3.reshape(1, 16),
      Wh[:D], Wh[D:], bh.reshape(1, 1))
    return out2[:, 0]


# speedup vs baseline: 2.4348x; 2.4348x over previous
import functools

import jax
import jax.numpy as jnp
from jax import lax
from jax.experimental import pallas as pl
from jax.experimental.pallas import tpu as pltpu
from jax.experimental.pallas import tpu_sc as plsc

B = 16384
D = 32
N_ROWS = 100000
NC, NS, L = 2, 16, 16
NW = NC * NS
HALF = B // 2

_mesh = plsc.VectorSubcoreMesh(core_axis_name="c", subcore_axis_name="s")


@functools.partial(
    pl.kernel,
    out_type=[jax.ShapeDtypeStruct((D, B), jnp.float32)] * 4,
    mesh=_mesh,
    compiler_params=pltpu.CompilerParams(needs_layout_passes=False),
    scratch_types=[
        pltpu.VMEM((N_ROWS,), jnp.float32),
        pltpu.VMEM((HALF,), jnp.int32),
        pltpu.VMEM((HALF,), jnp.float32),
    ],
)
def _sc_gather(u_hbm, i_hbm, ugT, igT, umT, imT,
               guT, giT, muT, miT,
               dimrow, idx, obuf):
    wid = lax.axis_index("s") * NC + lax.axis_index("c")

    def gath(k, carry):
        iv = idx[pl.ds(k * 16, 16)]
        obuf[pl.ds(k * 16, 16)] = plsc.load_gather(dimrow, [iv])
        return carry

    for tab, which, out in ((ugT, 0, guT), (umT, 0, muT),
                            (igT, 1, giT), (imT, 1, miT)):
        pltpu.sync_copy(tab.at[wid], dimrow)
        for h in range(2):
            pltpu.sync_copy((u_hbm if which == 0 else i_hbm)
                            .at[pl.ds(h * HALF, HALF)], idx)
            lax.fori_loop(0, HALF // 16, gath, 0, unroll=4)
            pltpu.sync_copy(obuf, out.at[wid, pl.ds(h * HALF, HALF)])


BLK = 2048


def _tc_mlp(gu, gi, mu, mi, w1u, w1i, b1, w2, b2, w3, b3, whg, whx, bh, out):
    f32 = jnp.float32
    x1 = jnp.maximum(jnp.dot(w1u[...], mu[...], preferred_element_type=f32)
                     + jnp.dot(w1i[...], mi[...], preferred_element_type=f32)
                     + b1[...], 0.0)
    x2 = jnp.maximum(jnp.dot(w2[...], x1, preferred_element_type=f32) + b2[...], 0.0)
    x3 = jnp.maximum(jnp.dot(w3[...], x2, preferred_element_type=f32) + b3[...], 0.0)
    gmf = gu[...] * gi[...]
    out[...] = (jnp.dot(whg[...], gmf, preferred_element_type=f32)
                + jnp.dot(whx[...], x3, preferred_element_type=f32)
                + bh[...])


def kernel(u, i, user_gmf, item_gmf, user_mlp, item_mlp,
           W1, b1, W2, b2, W3, b3, Wh, bh):
    u = u.astype(jnp.int32)
    i = i.astype(jnp.int32)
    gu, gi, mu, mi = _sc_gather(u, i, user_gmf.T, item_gmf.T,
                                user_mlp.T, item_mlp.T)

    col = lambda bs: pl.BlockSpec(bs, lambda b: (0, b))
    full = lambda bs: pl.BlockSpec(bs, lambda b: (0, 0))
    out2 = pl.pallas_call(
        _tc_mlp,
        grid=(B // BLK,),
        in_specs=[col((D, BLK))] * 4 + [
            full((64, D)), full((64, D)), full((64, 1)),
            full((32, 64)), full((32, 1)),
            full((16, 32)), full((16, 1)),
            full((1, D)), full((1, 16)), full((1, 1)),
        ],
        out_specs=col((1, BLK)),
        out_shape=jax.ShapeDtypeStruct((1, B), jnp.float32),
    )(gu, gi, mu, mi,
      W1[:D].T, W1[D:].T, b1.reshape(64, 1),
      W2.T, b2.reshape(32, 1),
      W3.T, b3.reshape(16, 1),
      Wh[:D].T, Wh[D:].T, bh.reshape(1, 1))
    return out2.reshape(B)
